# baseline (speedup 1.0000x reference)
import functools

import jax
import jax.numpy as jnp
from jax import lax
from jax.experimental import pallas as pl
from jax.experimental.pallas import tpu as pltpu
from jax.experimental.pallas import tpu_sc as plsc

B, N, C, MAX_INST = 16, 16384, 16, 16
NPW = N // 2
CHUNK = 2048
NCHUNK = NPW // CHUNK
GPC = CHUNK // 16
NGROUP = NPW // 16
UNROLL = 4

_F32 = jnp.float32
_I32 = jnp.int32


def _sqrt(x):
    xc = jnp.maximum(x, 1e-30)
    i = plsc.bitcast(xc, _I32)
    i = 0x5F3759DF - lax.shift_right_arithmetic(i, jnp.full((16,), 1, _I32))
    y = plsc.bitcast(i, _F32)
    xh = 0.5 * xc
    y = y * (1.5 - xh * y * y)
    y = y * (1.5 - xh * y * y)
    return x * y


@functools.partial(
    pl.kernel,
    out_type=jax.ShapeDtypeStruct((32, 16), _F32),
    mesh=plsc.VectorSubcoreMesh(core_axis_name="c", subcore_axis_name="s"),
    compiler_params=pltpu.CompilerParams(needs_layout_passes=False),
    scratch_types=[
        pltpu.VMEM((NPW,), _F32),
        pltpu.VMEM((NPW,), _F32),
        pltpu.VMEM((NPW,), _F32),
        pltpu.VMEM((NPW,), _I32),
        pltpu.VMEM((NPW,), _I32),
        pltpu.VMEM((CHUNK * C,), _F32),
        pltpu.VMEM((CHUNK * C,), _F32),
        pltpu.VMEM((64,), _F32),
        pltpu.VMEM((64,), _F32),
        pltpu.VMEM((48,), _F32),
        pltpu.VMEM((16,), _F32),
        pltpu.VMEM_SHARED((16, 64), _F32),
        pltpu.SemaphoreType.DMA,
        pltpu.SemaphoreType.DMA,
        pltpu.SemaphoreType.DMA,
    ],
)
def _loss_kernel(pts_hbm, sem_hbm, csv_hbm, label_hbm, out_hbm,
                 px_v, py_v, pz_v, cls_v, inst_v, sem0_v, sem1_v,
                 acc_v, pacc_v, ctr_v, out_v, shared,
                 dsem_in, dsem0, dsem1):
    c = lax.axis_index("c")
    s = lax.axis_index("s")
    f = c * 8 + s // 2
    h = s % 2
    wid = c * 16 + s
    base = h * NPW

    iota = jnp.arange(16, dtype=_I32)
    ones_f = jnp.ones((16,), _F32)

    sem_bufs = (sem0_v, sem1_v)
    dsems = (dsem0, dsem1)

    def fire_chunk(ck):
        buf, dsem = sem_bufs[ck % 2], dsems[ck % 2]
        return [
            pltpu.async_copy(
                sem_hbm.at[f, cc, pl.ds(base + ck * CHUNK, CHUNK)],
                buf.at[pl.ds(cc * CHUNK, CHUNK)], dsem)
            for cc in range(C)
        ]

    pending = {0: fire_chunk(0)}

    in_copies = [
        pltpu.async_copy(pts_hbm.at[0, f, pl.ds(base, NPW)], px_v, dsem_in),
        pltpu.async_copy(pts_hbm.at[1, f, pl.ds(base, NPW)], py_v, dsem_in),
        pltpu.async_copy(pts_hbm.at[2, f, pl.ds(base, NPW)], pz_v, dsem_in),
        pltpu.async_copy(label_hbm.at[f, 0, pl.ds(base, NPW)], cls_v, dsem_in),
        pltpu.async_copy(label_hbm.at[f, 1, pl.ds(base, NPW)], inst_v, dsem_in),
    ]

    for r in range(4):
        acc_v[pl.ds(r * 16, 16)] = jnp.zeros((16,), _F32)

    for cp in in_copies:
        cp.wait()

    acc_sem = jnp.zeros((16,), _F32)
    csv_copies = []
    for ck in range(NCHUNK):
        if ck + 1 < NCHUNK:
            pending[ck + 1] = fire_chunk(ck + 1)
        for cp in pending.pop(ck):
            cp.wait()
        if ck + 1 >= NCHUNK:
            csv_copies = [
                pltpu.async_copy(csv_hbm.at[i, f, pl.ds(base, NPW)],
                                 sem0_v.at[pl.ds(i * NPW, NPW)], dsem_in)
                for i in range(3)
            ]
        sem_v = sem_bufs[ck % 2]

        def group1(off, acc, sem_v=sem_v, nbase=None):
            cls = cls_v[pl.ds(off, 16)]
            inst = inst_v[pl.ds(off, 16)]
            gsem = plsc.load_gather(sem_v, [cls * CHUNK + nbase])
            px = px_v[pl.ds(off, 16)]
            py = py_v[pl.ds(off, 16)]
            pz = pz_v[pl.ds(off, 16)]
            plsc.addupdate_scatter(acc_v, [inst], px)
            plsc.addupdate_scatter(acc_v, [inst + 16], py)
            plsc.addupdate_scatter(acc_v, [inst + 32], pz)
            plsc.addupdate_scatter(acc_v, [inst + 48], ones_f)
            return acc + gsem

        def body1(g, acc, ck=ck, sem_v=sem_v):
            for u in range(UNROLL):
                loc = (g * UNROLL + u) * 16
                acc = group1(ck * CHUNK + loc, acc, sem_v, iota + loc)
            return acc

        acc_sem = lax.fori_loop(0, GPC // UNROLL, body1, acc_sem)

    pltpu.sync_copy(acc_v, shared.at[s])
    plsc.subcore_barrier()
    pltpu.sync_copy(shared.at[s ^ 1], pacc_v)

    cnt = acc_v[pl.ds(48, 16)] + pacc_v[pl.ds(48, 16)]
    inv = 1.0 / jnp.maximum(cnt, 1.0)
    for o in (0, 16, 32):
        ctr_v[pl.ds(o, 16)] = (acc_v[pl.ds(o, 16)] + pacc_v[pl.ds(o, 16)]) * inv

    for cp in csv_copies:
        cp.wait()

    def group2(off, acc):
        inst = inst_v[pl.ds(off, 16)]
        cx = plsc.load_gather(ctr_v, [inst])
        cy = plsc.load_gather(ctr_v, [inst + 16])
        cz = plsc.load_gather(ctr_v, [inst + 32])
        gtx = cx - px_v[pl.ds(off, 16)]
        gty = cy - py_v[pl.ds(off, 16)]
        gtz = cz - pz_v[pl.ds(off, 16)]
        w = jnp.minimum(_sqrt(gtx * gtx + gty * gty + gtz * gtz), 1.0)
        dx = gtx - sem0_v[pl.ds(off, 16)]
        dy = gty - sem0_v[pl.ds(NPW + off, 16)]
        dz = gtz - sem0_v[pl.ds(2 * NPW + off, 16)]
        d = _sqrt(dx * dx + dy * dy + dz * dz)
        return acc + d * w

    def body2(g, acc):
        for u in range(UNROLL):
            acc = group2((g * UNROLL + u) * 16, acc)
        return acc

    acc_csv = lax.fori_loop(0, NGROUP // UNROLL, body2,
                            jnp.zeros((16,), _F32))

    out_v[...] = (acc_csv * 0.2 - acc_sem) * (1.0 / (B * N))
    pltpu.sync_copy(out_v, out_hbm.at[wid])


def kernel(points, pred_sem_mat, pred_center_shift_vectors, label, device):
    partials = _loss_kernel(
        jnp.transpose(points, (2, 0, 1)),
        jnp.transpose(pred_sem_mat, (0, 2, 1)),
        jnp.transpose(pred_center_shift_vectors, (2, 0, 1)),
        label,
    )
    return jnp.sum(partials)


# speedup vs baseline: 1.0063x; 1.0063x over previous
import functools

import jax
import jax.numpy as jnp
from jax import lax
from jax.experimental import pallas as pl
from jax.experimental.pallas import tpu as pltpu
from jax.experimental.pallas import tpu_sc as plsc

B, N, C, MAX_INST = 16, 16384, 16, 16
NPW = N // 2
CHUNK = 2048
NCHUNK = NPW // CHUNK
GPC = CHUNK // 16
NGROUP = NPW // 16
UNROLL = 2

_F32 = jnp.float32
_I32 = jnp.int32


def _sqrt(x):
    xc = jnp.maximum(x, 1e-30)
    i = plsc.bitcast(xc, _I32)
    i = 0x5F3759DF - lax.shift_right_arithmetic(i, jnp.full((16,), 1, _I32))
    y = plsc.bitcast(i, _F32)
    xh = 0.5 * xc
    y = y * (1.5 - xh * y * y)
    y = y * (1.5 - xh * y * y)
    return x * y


@functools.partial(
    pl.kernel,
    out_type=jax.ShapeDtypeStruct((32, 16), _F32),
    mesh=plsc.VectorSubcoreMesh(core_axis_name="c", subcore_axis_name="s"),
    compiler_params=pltpu.CompilerParams(needs_layout_passes=False,
                                         skip_device_barrier=True),
    scratch_types=[
        pltpu.VMEM((NPW,), _F32),
        pltpu.VMEM((NPW,), _F32),
        pltpu.VMEM((NPW,), _F32),
        pltpu.VMEM((NPW,), _I32),
        pltpu.VMEM((NPW,), _I32),
        pltpu.VMEM((CHUNK * C,), _F32),
        pltpu.VMEM((CHUNK * C,), _F32),
        pltpu.VMEM((64,), _F32),
        pltpu.VMEM((64,), _F32),
        pltpu.VMEM((48,), _F32),
        pltpu.VMEM((16,), _F32),
        pltpu.VMEM_SHARED((16, 64), _F32),
        pltpu.SemaphoreType.DMA,
        pltpu.SemaphoreType.DMA,
        pltpu.SemaphoreType.DMA,
    ],
)
def _loss_kernel(pts_hbm, sem_hbm, csv_hbm, label_hbm, out_hbm,
                 px_v, py_v, pz_v, cls_v, inst_v, sem0_v, sem1_v,
                 acc_v, pacc_v, ctr_v, out_v, shared,
                 dsem_in, dsem0, dsem1):
    c = lax.axis_index("c")
    s = lax.axis_index("s")
    f = c * 8 + s // 2
    h = s % 2
    wid = c * 16 + s
    base = h * NPW

    iota = jnp.arange(16, dtype=_I32)
    ones_f = jnp.ones((16,), _F32)

    sem_bufs = (sem0_v, sem1_v)
    dsems = (dsem0, dsem1)

    def fire_chunk(ck):
        buf, dsem = sem_bufs[ck % 2], dsems[ck % 2]
        return [
            pltpu.async_copy(
                sem_hbm.at[f, cc, pl.ds(base + ck * CHUNK, CHUNK)],
                buf.at[pl.ds(cc * CHUNK, CHUNK)], dsem)
            for cc in range(C)
        ]

    pending = {0: fire_chunk(0)}

    in_copies = [
        pltpu.async_copy(pts_hbm.at[0, f, pl.ds(base, NPW)], px_v, dsem_in),
        pltpu.async_copy(pts_hbm.at[1, f, pl.ds(base, NPW)], py_v, dsem_in),
        pltpu.async_copy(pts_hbm.at[2, f, pl.ds(base, NPW)], pz_v, dsem_in),
        pltpu.async_copy(label_hbm.at[f, 0, pl.ds(base, NPW)], cls_v, dsem_in),
        pltpu.async_copy(label_hbm.at[f, 1, pl.ds(base, NPW)], inst_v, dsem_in),
    ]

    for r in range(4):
        acc_v[pl.ds(r * 16, 16)] = jnp.zeros((16,), _F32)

    for cp in in_copies:
        cp.wait()

    acc_sem = jnp.zeros((16,), _F32)
    csv_copies = []
    for ck in range(NCHUNK):
        if ck + 1 < NCHUNK:
            pending[ck + 1] = fire_chunk(ck + 1)
        for cp in pending.pop(ck):
            cp.wait()
        if ck + 1 >= NCHUNK:
            csv_copies = [
                pltpu.async_copy(csv_hbm.at[i, f, pl.ds(base, NPW)],
                                 sem0_v.at[pl.ds(i * NPW, NPW)], dsem_in)
                for i in range(3)
            ]
        sem_v = sem_bufs[ck % 2]

        def group1(off, acc, sem_v=sem_v, nbase=None):
            cls = cls_v[pl.ds(off, 16)]
            inst = inst_v[pl.ds(off, 16)]
            gsem = plsc.load_gather(sem_v, [cls * CHUNK + nbase])
            px = px_v[pl.ds(off, 16)]
            py = py_v[pl.ds(off, 16)]
            pz = pz_v[pl.ds(off, 16)]
            plsc.addupdate_scatter(acc_v, [inst], px)
            plsc.addupdate_scatter(acc_v, [inst + 16], py)
            plsc.addupdate_scatter(acc_v, [inst + 32], pz)
            plsc.addupdate_scatter(acc_v, [inst + 48], ones_f)
            return acc + gsem

        def body1(g, acc, ck=ck, sem_v=sem_v):
            for u in range(UNROLL):
                loc = (g * UNROLL + u) * 16
                acc = group1(ck * CHUNK + loc, acc, sem_v, iota + loc)
            return acc

        acc_sem = lax.fori_loop(0, GPC // UNROLL, body1, acc_sem)

    pltpu.sync_copy(acc_v, shared.at[s])
    plsc.subcore_barrier()
    pltpu.sync_copy(shared.at[s ^ 1], pacc_v)

    cnt = acc_v[pl.ds(48, 16)] + pacc_v[pl.ds(48, 16)]
    inv = 1.0 / jnp.maximum(cnt, 1.0)
    for o in (0, 16, 32):
        ctr_v[pl.ds(o, 16)] = (acc_v[pl.ds(o, 16)] + pacc_v[pl.ds(o, 16)]) * inv

    for cp in csv_copies:
        cp.wait()

    def group2(off, acc):
        inst = inst_v[pl.ds(off, 16)]
        cx = plsc.load_gather(ctr_v, [inst])
        cy = plsc.load_gather(ctr_v, [inst + 16])
        cz = plsc.load_gather(ctr_v, [inst + 32])
        gtx = cx - px_v[pl.ds(off, 16)]
        gty = cy - py_v[pl.ds(off, 16)]
        gtz = cz - pz_v[pl.ds(off, 16)]
        w = jnp.minimum(_sqrt(gtx * gtx + gty * gty + gtz * gtz), 1.0)
        dx = gtx - sem0_v[pl.ds(off, 16)]
        dy = gty - sem0_v[pl.ds(NPW + off, 16)]
        dz = gtz - sem0_v[pl.ds(2 * NPW + off, 16)]
        d = _sqrt(dx * dx + dy * dy + dz * dz)
        return acc + d * w

    def body2(g, acc):
        for u in range(UNROLL):
            acc = group2((g * UNROLL + u) * 16, acc)
        return acc

    acc_csv = lax.fori_loop(0, NGROUP // UNROLL, body2,
                            jnp.zeros((16,), _F32))

    out_v[...] = (acc_csv * 0.2 - acc_sem) * (1.0 / (B * N))
    pltpu.sync_copy(out_v, out_hbm.at[wid])


def kernel(points, pred_sem_mat, pred_center_shift_vectors, label, device):
    partials = _loss_kernel(
        jnp.transpose(points, (2, 0, 1)),
        jnp.transpose(pred_sem_mat, (0, 2, 1)),
        jnp.transpose(pred_center_shift_vectors, (2, 0, 1)),
        label,
    )
    return jnp.sum(partials)


# speedup vs baseline: 1.0140x; 1.0077x over previous
import functools

import jax
import jax.numpy as jnp
from jax import lax
from jax.experimental import pallas as pl
from jax.experimental.pallas import tpu as pltpu
from jax.experimental.pallas import tpu_sc as plsc

B, N, C, MAX_INST = 16, 16384, 16, 16
NPW = N // 2
CHUNK = 2048
NCHUNK = NPW // CHUNK
GPC = CHUNK // 16
NGROUP = NPW // 16
UNROLL = 2

_F32 = jnp.float32
_I32 = jnp.int32


def _sqrt(x):
    xc = jnp.maximum(x, 1e-30)
    i = plsc.bitcast(xc, _I32)
    i = 0x5F3759DF - lax.shift_right_arithmetic(i, jnp.full((16,), 1, _I32))
    y = plsc.bitcast(i, _F32)
    xh = 0.5 * xc
    y = y * (1.5 - xh * y * y)
    y = y * (1.5 - xh * y * y)
    return x * y


@functools.partial(
    pl.kernel,
    out_type=jax.ShapeDtypeStruct((32, 16), _F32),
    mesh=plsc.VectorSubcoreMesh(core_axis_name="c", subcore_axis_name="s"),
    compiler_params=pltpu.CompilerParams(needs_layout_passes=False,
                                         skip_device_barrier=True),
    scratch_types=[
        pltpu.VMEM((NPW,), _F32),
        pltpu.VMEM((NPW,), _F32),
        pltpu.VMEM((NPW,), _F32),
        pltpu.VMEM((NPW,), _I32),
        pltpu.VMEM((NPW,), _I32),
        pltpu.VMEM((CHUNK * C,), _F32),
        pltpu.VMEM((CHUNK * C,), _F32),
        pltpu.VMEM((256,), _F32),
        pltpu.VMEM((256,), _F32),
        pltpu.VMEM((256,), _F32),
        pltpu.VMEM((256,), _F32),
        pltpu.VMEM((64,), _F32),
        pltpu.VMEM((64,), _F32),
        pltpu.VMEM((48,), _F32),
        pltpu.VMEM((16,), _F32),
        pltpu.VMEM_SHARED((16, 64), _F32),
        pltpu.SemaphoreType.DMA,
        pltpu.SemaphoreType.DMA,
        pltpu.SemaphoreType.DMA,
    ],
)
def _loss_kernel(pts_hbm, sem_hbm, csv_hbm, label_hbm, out_hbm,
                 px_v, py_v, pz_v, cls_v, inst_v, sem0_v, sem1_v,
                 accx_v, accy_v, accz_v, accn_v,
                 acc_v, pacc_v, ctr_v, out_v, shared,
                 dsem_in, dsem0, dsem1):
    c = lax.axis_index("c")
    s = lax.axis_index("s")
    f = c * 8 + s // 2
    h = s % 2
    wid = c * 16 + s
    base = h * NPW

    iota = jnp.arange(16, dtype=_I32)
    iota16 = iota * 16
    ones_f = jnp.ones((16,), _F32)

    sem_bufs = (sem0_v, sem1_v)
    dsems = (dsem0, dsem1)

    def fire_chunk(ck):
        buf, dsem = sem_bufs[ck % 2], dsems[ck % 2]
        return [
            pltpu.async_copy(
                sem_hbm.at[f, cc, pl.ds(base + ck * CHUNK, CHUNK)],
                buf.at[pl.ds(cc * CHUNK, CHUNK)], dsem)
            for cc in range(C)
        ]

    pending = {0: fire_chunk(0)}

    in_copies = [
        pltpu.async_copy(pts_hbm.at[0, f, pl.ds(base, NPW)], px_v, dsem_in),
        pltpu.async_copy(pts_hbm.at[1, f, pl.ds(base, NPW)], py_v, dsem_in),
        pltpu.async_copy(pts_hbm.at[2, f, pl.ds(base, NPW)], pz_v, dsem_in),
        pltpu.async_copy(label_hbm.at[f, 0, pl.ds(base, NPW)], cls_v, dsem_in),
        pltpu.async_copy(label_hbm.at[f, 1, pl.ds(base, NPW)], inst_v, dsem_in),
    ]

    zero_f = jnp.zeros((16,), _F32)
    for r in range(16):
        accx_v[pl.ds(r * 16, 16)] = zero_f
        accy_v[pl.ds(r * 16, 16)] = zero_f
        accz_v[pl.ds(r * 16, 16)] = zero_f
        accn_v[pl.ds(r * 16, 16)] = zero_f

    for cp in in_copies:
        cp.wait()

    acc_sem = jnp.zeros((16,), _F32)
    csv_copies = []
    for ck in range(NCHUNK):
        if ck + 1 < NCHUNK:
            pending[ck + 1] = fire_chunk(ck + 1)
        for cp in pending.pop(ck):
            cp.wait()
        if ck + 1 >= NCHUNK:
            csv_copies = [
                pltpu.async_copy(csv_hbm.at[i, f, pl.ds(base, NPW)],
                                 sem0_v.at[pl.ds(i * NPW, NPW)], dsem_in)
                for i in range(3)
            ]
        sem_v = sem_bufs[ck % 2]

        def group1(off, acc, sem_v=sem_v, nbase=None):
            cls = cls_v[pl.ds(off, 16)]
            inst = inst_v[pl.ds(off, 16)]
            gsem = plsc.load_gather(sem_v, [cls * CHUNK + nbase])
            px = px_v[pl.ds(off, 16)]
            py = py_v[pl.ds(off, 16)]
            pz = pz_v[pl.ds(off, 16)]
            lidx = iota16 + inst
            plsc.addupdate_scatter(accx_v, [lidx], px)
            plsc.addupdate_scatter(accy_v, [lidx], py)
            plsc.addupdate_scatter(accz_v, [lidx], pz)
            plsc.addupdate_scatter(accn_v, [lidx], ones_f)
            return acc + gsem

        def body1(g, acc, ck=ck, sem_v=sem_v):
            for u in range(UNROLL):
                loc = (g * UNROLL + u) * 16
                acc = group1(ck * CHUNK + loc, acc, sem_v, iota + loc)
            return acc

        acc_sem = lax.fori_loop(0, GPC // UNROLL, body1, acc_sem)

    for dst, ref in ((0, accx_v), (16, accy_v), (32, accz_v), (48, accn_v)):
        tot = ref[pl.ds(0, 16)]
        for l in range(1, 16):
            tot = tot + ref[pl.ds(l * 16, 16)]
        acc_v[pl.ds(dst, 16)] = tot

    pltpu.sync_copy(acc_v, shared.at[s])
    plsc.subcore_barrier()
    pltpu.sync_copy(shared.at[s ^ 1], pacc_v)

    cnt = acc_v[pl.ds(48, 16)] + pacc_v[pl.ds(48, 16)]
    inv = 1.0 / jnp.maximum(cnt, 1.0)
    for o in (0, 16, 32):
        ctr_v[pl.ds(o, 16)] = (acc_v[pl.ds(o, 16)] + pacc_v[pl.ds(o, 16)]) * inv

    for cp in csv_copies:
        cp.wait()

    def group2(off, acc):
        inst = inst_v[pl.ds(off, 16)]
        cx = plsc.load_gather(ctr_v, [inst])
        cy = plsc.load_gather(ctr_v, [inst + 16])
        cz = plsc.load_gather(ctr_v, [inst + 32])
        gtx = cx - px_v[pl.ds(off, 16)]
        gty = cy - py_v[pl.ds(off, 16)]
        gtz = cz - pz_v[pl.ds(off, 16)]
        w = jnp.minimum(_sqrt(gtx * gtx + gty * gty + gtz * gtz), 1.0)
        dx = gtx - sem0_v[pl.ds(off, 16)]
        dy = gty - sem0_v[pl.ds(NPW + off, 16)]
        dz = gtz - sem0_v[pl.ds(2 * NPW + off, 16)]
        d = _sqrt(dx * dx + dy * dy + dz * dz)
        return acc + d * w

    def body2(g, acc):
        for u in range(UNROLL):
            acc = group2((g * UNROLL + u) * 16, acc)
        return acc

    acc_csv = lax.fori_loop(0, NGROUP // UNROLL, body2,
                            jnp.zeros((16,), _F32))

    out_v[...] = (acc_csv * 0.2 - acc_sem) * (1.0 / (B * N))
    pltpu.sync_copy(out_v, out_hbm.at[wid])


def kernel(points, pred_sem_mat, pred_center_shift_vectors, label, device):
    partials = _loss_kernel(
        jnp.transpose(points, (2, 0, 1)),
        jnp.transpose(pred_sem_mat, (0, 2, 1)),
        jnp.transpose(pred_center_shift_vectors, (2, 0, 1)),
        label,
    )
    return jnp.sum(partials)
